# Initial kernel scaffold; baseline (speedup 1.0000x reference)
import jax
import jax.numpy as jnp
from jax.experimental import pallas as pl


def kernel(x, edge_index, W1, b1, W2, b2):
    raise NotImplementedError("write your pallas kernel here")


# speedup vs baseline: 53.2054x; 53.2054x over previous
import functools

import jax
import jax.numpy as jnp
from jax import lax
from jax.experimental import pallas as pl
from jax.experimental.pallas import tpu as pltpu
from jax.experimental.pallas import tpu_sc as plsc

NC = 2
NS = 16
LANES = 128
CH = 8


def _round_up(a, b):
    return (a + b - 1) // b * b


def _sc_mesh():
    return plsc.VectorSubcoreMesh(
        core_axis_name="c", subcore_axis_name="s", num_cores=NC, num_subcores=NS
    )


_SC_PARAMS = pltpu.CompilerParams(use_tc_tiling_on_sc=False)


def _make_deg_kernel(NP, ROWS):
    RW = ROWS // (NC * NS)
    NB = RW // CH
    RB = NP // NS

    @functools.partial(
        pl.kernel,
        out_type=jax.ShapeDtypeStruct((NC * NP,), jnp.float32),
        mesh=_sc_mesh(),
        scratch_types=[
            pltpu.VMEM((CH, LANES), jnp.int32),
            pltpu.VMEM((LANES,), jnp.float32),
            pltpu.VMEM_SHARED((NP,), jnp.float32),
            pltpu.SemaphoreType.DMA,
        ],
        compiler_params=_SC_PARAMS,
    )
    def deg_kernel(dst2_hbm, z1_hbm, out_hbm, idx_v, ones_v, acc, ssem):
        c = lax.axis_index("c")
        s = lax.axis_index("s")
        wid = s * NC + c
        pltpu.sync_copy(z1_hbm.at[pl.ds(s * RB, RB)], acc.at[pl.ds(s * RB, RB)])
        for k in range(LANES // 16):
            ones_v[pl.ds(k * 16, 16)] = jnp.full((16,), 1.0, jnp.float32)
        plsc.subcore_barrier()

        base = wid * RW

        @pl.loop(0, NB)
        def _(b):
            pltpu.sync_copy(dst2_hbm.at[pl.ds(base + b * CH, CH)], idx_v)
            descs = []
            for j in range(CH):
                descs.append(
                    pltpu.async_copy(ones_v, acc.at[idx_v.at[j]], ssem, add=True)
                )
            for d in descs:
                d.wait()

        plsc.subcore_barrier()
        pltpu.sync_copy(
            acc.at[pl.ds(s * RB, RB)], out_hbm.at[pl.ds(c * NP + s * RB, RB)]
        )

    return deg_kernel


def _make_agg_kernel(NP, ROWS, D):
    RW = ROWS // (NC * NS)
    NB = RW // CH
    RB = NP // NS

    @functools.partial(
        pl.kernel,
        out_type=jax.ShapeDtypeStruct((NC * NP, D), jnp.float32),
        mesh=_sc_mesh(),
        scratch_types=[
            pltpu.VMEM((CH, LANES), jnp.int32),
            pltpu.VMEM((CH, LANES), jnp.int32),
            pltpu.VMEM((CH, LANES, D), jnp.float32),
            pltpu.VMEM_SHARED((NP, D), jnp.float32),
            pltpu.SemaphoreType.DMA,
            pltpu.SemaphoreType.DMA,
        ],
        compiler_params=_SC_PARAMS,
    )
    def agg_kernel(src2_hbm, dst2_hbm, u_hbm, z_hbm, out_hbm,
                   isrc, idst, rows, acc, gsem, ssem):
        c = lax.axis_index("c")
        s = lax.axis_index("s")
        wid = s * NC + c
        pltpu.sync_copy(z_hbm.at[pl.ds(s * RB, RB)], acc.at[pl.ds(s * RB, RB)])
        plsc.subcore_barrier()

        base = wid * RW

        @pl.loop(0, NB)
        def _(b):
            pltpu.sync_copy(src2_hbm.at[pl.ds(base + b * CH, CH)], isrc)
            pltpu.sync_copy(dst2_hbm.at[pl.ds(base + b * CH, CH)], idst)
            gd = []
            for j in range(CH):
                gd.append(
                    pltpu.async_copy(u_hbm.at[isrc.at[j]], rows.at[j], gsem)
                )
            sd = []
            for j in range(CH):
                gd[j].wait()
                sd.append(
                    pltpu.async_copy(rows.at[j], acc.at[idst.at[j]], ssem,
                                     add=True)
                )
            for d in sd:
                d.wait()

        plsc.subcore_barrier()
        pltpu.sync_copy(
            acc.at[pl.ds(s * RB, RB)], out_hbm.at[pl.ds(c * NP + s * RB, RB)]
        )

    return agg_kernel


def _tc1_body(d0, d1, x, w, o):
    dv = lax.rsqrt(d0[...] + d1[...] + 1.0)
    o[...] = jnp.dot(x[...], w[...], preferred_element_type=jnp.float32) * dv


def _tc2_body(d0, d1, a0, a1, u1, w, b, o):
    dv = lax.rsqrt(d0[...] + d1[...] + 1.0)
    t = dv * (a0[...] + a1[...] + u1[...]) + b[...]
    r = jnp.maximum(t, 0.0)
    o[...] = jnp.dot(r, w[...], preferred_element_type=jnp.float32) * dv


def _tc3_body(d0, d1, a0, a1, u2, b, o):
    dv = lax.rsqrt(d0[...] + d1[...] + 1.0)
    o[...] = dv * (a0[...] + a1[...] + u2[...]) + b[...]


def _col(bn):
    return pl.BlockSpec((bn, 1), lambda i: (i, 0))


def _mat(bn, d):
    return pl.BlockSpec((bn, d), lambda i: (i, 0))


def _full(shape):
    return pl.BlockSpec(shape, lambda i: tuple(0 for _ in shape))


def kernel(x, edge_index, W1, b1, W2, b2):
    N, IN_D = x.shape
    HID = W1.shape[1]
    OUT = W2.shape[1]
    E = edge_index.shape[1]

    BN = 3584
    NP = _round_up(N + 1, max(BN, NS * 8))
    GRID = NP // BN
    EP = _round_up(E, NC * NS * CH * LANES)
    ROWS = EP // LANES

    f32 = jnp.float32
    x_pad = jnp.concatenate([x.astype(f32), jnp.zeros((NP - N, IN_D), f32)])
    pad = jnp.full((2, EP - E), N, jnp.int32)
    ep = jnp.concatenate([edge_index.astype(jnp.int32), pad], axis=1)
    src2 = ep[0].reshape(ROWS, LANES)
    dst2 = ep[1].reshape(ROWS, LANES)
    z1 = jnp.zeros((NP,), f32)
    zh = jnp.zeros((NP, HID), f32)
    zo = jnp.zeros((NP, OUT), f32)

    deg = _make_deg_kernel(NP, ROWS)(dst2, z1)
    d0 = deg[:NP].reshape(NP, 1)
    d1 = deg[NP:].reshape(NP, 1)

    u1 = pl.pallas_call(
        _tc1_body,
        grid=(GRID,),
        in_specs=[_col(BN), _col(BN), _mat(BN, IN_D), _full((IN_D, HID))],
        out_specs=_mat(BN, HID),
        out_shape=jax.ShapeDtypeStruct((NP, HID), f32),
    )(d0, d1, x_pad, W1.astype(f32))

    agg1 = _make_agg_kernel(NP, ROWS, HID)(src2, dst2, u1, zh)
    a0, a1 = agg1[:NP], agg1[NP:]

    u2 = pl.pallas_call(
        _tc2_body,
        grid=(GRID,),
        in_specs=[_col(BN), _col(BN), _mat(BN, HID), _mat(BN, HID),
                  _mat(BN, HID), _full((HID, OUT)), _full((1, HID))],
        out_specs=_mat(BN, OUT),
        out_shape=jax.ShapeDtypeStruct((NP, OUT), f32),
    )(d0, d1, a0, a1, u1, W2.astype(f32), b1.astype(f32).reshape(1, HID))

    agg2 = _make_agg_kernel(NP, ROWS, OUT)(src2, dst2, u2, zo)
    c0, c1 = agg2[:NP], agg2[NP:]

    out = pl.pallas_call(
        _tc3_body,
        grid=(GRID,),
        in_specs=[_col(BN), _col(BN), _mat(BN, OUT), _mat(BN, OUT),
                  _mat(BN, OUT), _full((1, OUT))],
        out_specs=_mat(BN, OUT),
        out_shape=jax.ShapeDtypeStruct((NP, OUT), f32),
    )(d0, d1, c0, c1, u2, b2.astype(f32).reshape(1, OUT))

    return out[:N]


# speedup vs baseline: 67.4255x; 1.2673x over previous
import functools

import jax
import jax.numpy as jnp
from jax import lax
from jax.experimental import pallas as pl
from jax.experimental.pallas import tpu as pltpu
from jax.experimental.pallas import tpu_sc as plsc

NC = 2
NS = 16
LANES = 128
CHA = 4
CHD = 8


def _round_up(a, b):
    return (a + b - 1) // b * b


def _sc_mesh():
    return plsc.VectorSubcoreMesh(
        core_axis_name="c", subcore_axis_name="s", num_cores=NC, num_subcores=NS
    )


_SC_PARAMS = pltpu.CompilerParams(use_tc_tiling_on_sc=False)


def _make_deg_kernel(NP, ROWS):
    RW = ROWS // (NC * NS)
    NB = RW // CHD
    RB = NP // NS

    @functools.partial(
        pl.kernel,
        out_type=jax.ShapeDtypeStruct((NC * NP,), jnp.float32),
        mesh=_sc_mesh(),
        scratch_types=[
            pltpu.VMEM((2, CHD, LANES), jnp.int32),
            pltpu.VMEM((LANES,), jnp.float32),
            pltpu.VMEM_SHARED((NP,), jnp.float32),
            pltpu.SemaphoreType.DMA,
            pltpu.SemaphoreType.DMA,
        ],
        compiler_params=_SC_PARAMS,
    )
    def deg_kernel(dst2_hbm, z1_hbm, out_hbm, idx, ones_v, acc, lsem, ssem):
        c = lax.axis_index("c")
        s = lax.axis_index("s")
        wid = s * NC + c
        pltpu.sync_copy(z1_hbm.at[pl.ds(s * RB, RB)], acc.at[pl.ds(s * RB, RB)])
        for k in range(LANES // 16):
            ones_v[pl.ds(k * 16, 16)] = jnp.full((16,), 1.0, jnp.float32)
        plsc.subcore_barrier()

        base = wid * RW
        pltpu.sync_copy(dst2_hbm.at[pl.ds(base, CHD)], idx.at[0])

        @pl.loop(0, NB // 2)
        def _(t):
            for p in (0, 1):
                b = 2 * t + p
                q = 1 - p

                @pl.when(b > 0)
                def _():
                    for j in range(CHD):
                        pltpu.make_async_copy(
                            ones_v, acc.at[idx.at[q, j]], ssem
                        ).wait()

                @pl.when(b + 1 < NB)
                def _():
                    pltpu.async_copy(
                        dst2_hbm.at[pl.ds(base + (b + 1) * CHD, CHD)],
                        idx.at[q], lsem,
                    )

                for j in range(CHD):
                    pltpu.async_copy(
                        ones_v, acc.at[idx.at[p, j]], ssem, add=True
                    )

                @pl.when(b + 1 < NB)
                def _():
                    pltpu.make_async_copy(
                        dst2_hbm.at[pl.ds(0, CHD)], idx.at[q], lsem
                    ).wait()

        for j in range(CHD):
            pltpu.make_async_copy(ones_v, acc.at[idx.at[1, j]], ssem).wait()

        plsc.subcore_barrier()
        pltpu.sync_copy(
            acc.at[pl.ds(s * RB, RB)], out_hbm.at[pl.ds(c * NP + s * RB, RB)]
        )

    return deg_kernel


def _make_agg_kernel(NP, ROWS, D):
    RW = ROWS // (NC * NS)
    NB = RW // CHA
    RB = NP // NS

    @functools.partial(
        pl.kernel,
        out_type=jax.ShapeDtypeStruct((NC * NP, D), jnp.float32),
        mesh=_sc_mesh(),
        scratch_types=[
            pltpu.VMEM((2, CHA, LANES), jnp.int32),
            pltpu.VMEM((2, CHA, LANES), jnp.int32),
            pltpu.VMEM((2, CHA, LANES, D), jnp.float32),
            pltpu.VMEM_SHARED((NP, D), jnp.float32),
            pltpu.SemaphoreType.DMA,
            pltpu.SemaphoreType.DMA,
            pltpu.SemaphoreType.DMA,
        ],
        compiler_params=_SC_PARAMS,
    )
    def agg_kernel(src2_hbm, dst2_hbm, u_hbm, z_hbm, out_hbm,
                   isrc, idst, rows, acc, lsem, gsem, ssem):
        c = lax.axis_index("c")
        s = lax.axis_index("s")
        wid = s * NC + c
        pltpu.sync_copy(z_hbm.at[pl.ds(s * RB, RB)], acc.at[pl.ds(s * RB, RB)])
        plsc.subcore_barrier()

        base = wid * RW
        pltpu.sync_copy(src2_hbm.at[pl.ds(base, CHA)], isrc.at[0])
        pltpu.sync_copy(dst2_hbm.at[pl.ds(base, CHA)], idst.at[0])
        for j in range(CHA):
            pltpu.async_copy(u_hbm.at[isrc.at[0, j]], rows.at[0, j], gsem)

        @pl.loop(0, NB // 2)
        def _(t):
            for p in (0, 1):
                b = 2 * t + p
                q = 1 - p

                @pl.when(b > 0)
                def _():
                    for j in range(CHA):
                        pltpu.make_async_copy(
                            rows.at[q, j], acc.at[idst.at[q, j]], ssem
                        ).wait()

                @pl.when(b + 1 < NB)
                def _():
                    pltpu.async_copy(
                        src2_hbm.at[pl.ds(base + (b + 1) * CHA, CHA)],
                        isrc.at[q], lsem,
                    )
                    pltpu.async_copy(
                        dst2_hbm.at[pl.ds(base + (b + 1) * CHA, CHA)],
                        idst.at[q], lsem,
                    )

                for j in range(CHA):
                    pltpu.make_async_copy(
                        u_hbm.at[isrc.at[p, j]], rows.at[p, j], gsem
                    ).wait()
                for j in range(CHA):
                    pltpu.async_copy(
                        rows.at[p, j], acc.at[idst.at[p, j]], ssem, add=True
                    )

                @pl.when(b + 1 < NB)
                def _():
                    pltpu.make_async_copy(
                        src2_hbm.at[pl.ds(0, CHA)], isrc.at[q], lsem
                    ).wait()
                    pltpu.make_async_copy(
                        dst2_hbm.at[pl.ds(0, CHA)], idst.at[q], lsem
                    ).wait()
                    for j in range(CHA):
                        pltpu.async_copy(
                            u_hbm.at[isrc.at[q, j]], rows.at[q, j], gsem
                        )

        for j in range(CHA):
            pltpu.make_async_copy(
                rows.at[1, j], acc.at[idst.at[1, j]], ssem
            ).wait()

        plsc.subcore_barrier()
        pltpu.sync_copy(
            acc.at[pl.ds(s * RB, RB)], out_hbm.at[pl.ds(c * NP + s * RB, RB)]
        )

    return agg_kernel


def _tc1_body(d0, d1, x, w, o):
    dv = lax.rsqrt(d0[...] + d1[...] + 1.0)
    o[...] = jnp.dot(x[...], w[...], preferred_element_type=jnp.float32) * dv


def _tc2_body(d0, d1, a0, a1, u1, w, b, o):
    dv = lax.rsqrt(d0[...] + d1[...] + 1.0)
    t = dv * (a0[...] + a1[...] + u1[...]) + b[...]
    r = jnp.maximum(t, 0.0)
    o[...] = jnp.dot(r, w[...], preferred_element_type=jnp.float32) * dv


def _tc3_body(d0, d1, a0, a1, u2, b, o):
    dv = lax.rsqrt(d0[...] + d1[...] + 1.0)
    o[...] = dv * (a0[...] + a1[...] + u2[...]) + b[...]


def _half(bn, d, off):
    return pl.BlockSpec((bn, d), lambda i, o=off: (i + o, 0))


def _mat(bn, d):
    return pl.BlockSpec((bn, d), lambda i: (i, 0))


def _full(shape):
    return pl.BlockSpec(shape, lambda i: tuple(0 for _ in shape))


def kernel(x, edge_index, W1, b1, W2, b2):
    N, IN_D = x.shape
    HID = W1.shape[1]
    OUT = W2.shape[1]
    E = edge_index.shape[1]

    BN = 3584
    NP = _round_up(N + 1, max(BN, NS * 8))
    GRID = NP // BN
    EP = _round_up(E, NC * NS * 2 * max(CHA, CHD) * LANES)
    ROWS = EP // LANES

    f32 = jnp.float32
    pad = jnp.full((2, EP - E), N, jnp.int32)
    ep = jnp.concatenate([edge_index.astype(jnp.int32), pad], axis=1)
    src2 = ep[0].reshape(ROWS, LANES)
    dst2 = ep[1].reshape(ROWS, LANES)
    z1 = jnp.zeros((NP,), f32)
    zh = jnp.zeros((NP, HID), f32)
    zo = jnp.zeros((NP, OUT), f32)

    deg = _make_deg_kernel(NP, ROWS)(dst2, z1)
    deg2 = deg.reshape(NC * NP, 1)

    u1 = pl.pallas_call(
        _tc1_body,
        grid=(GRID,),
        in_specs=[_half(BN, 1, 0), _half(BN, 1, GRID), _mat(BN, IN_D),
                  _full((IN_D, HID))],
        out_specs=_mat(BN, HID),
        out_shape=jax.ShapeDtypeStruct((NP, HID), f32),
    )(deg2, deg2, x.astype(f32), W1.astype(f32))

    agg1 = _make_agg_kernel(NP, ROWS, HID)(src2, dst2, u1, zh)

    u2 = pl.pallas_call(
        _tc2_body,
        grid=(GRID,),
        in_specs=[_half(BN, 1, 0), _half(BN, 1, GRID),
                  _half(BN, HID, 0), _half(BN, HID, GRID), _mat(BN, HID),
                  _full((HID, OUT)), _full((1, HID))],
        out_specs=_mat(BN, OUT),
        out_shape=jax.ShapeDtypeStruct((NP, OUT), f32),
    )(deg2, deg2, agg1, agg1, u1, W2.astype(f32),
      b1.astype(f32).reshape(1, HID))

    agg2 = _make_agg_kernel(NP, ROWS, OUT)(src2, dst2, u2, zo)

    out = pl.pallas_call(
        _tc3_body,
        grid=(GRID,),
        in_specs=[_half(BN, 1, 0), _half(BN, 1, GRID),
                  _half(BN, OUT, 0), _half(BN, OUT, GRID), _mat(BN, OUT),
                  _full((1, OUT))],
        out_specs=_mat(BN, OUT),
        out_shape=jax.ShapeDtypeStruct((N, OUT), f32),
    )(deg2, deg2, agg2, agg2, u2, b2.astype(f32).reshape(1, OUT))

    return out


# speedup vs baseline: 87.1909x; 1.2931x over previous
import functools

import jax
import jax.numpy as jnp
from jax import lax
from jax.experimental import pallas as pl
from jax.experimental.pallas import tpu as pltpu
from jax.experimental.pallas import tpu_sc as plsc

NC = 2
NS = 16
LANES = 128
CHA = 4
CHD = 4


def _round_up(a, b):
    return (a + b - 1) // b * b


def _sc_mesh():
    return plsc.VectorSubcoreMesh(
        core_axis_name="c", subcore_axis_name="s", num_cores=NC, num_subcores=NS
    )


_SC_PARAMS = pltpu.CompilerParams(use_tc_tiling_on_sc=False)


def _make_deg_kernel(NP, ROWS, D):
    RW = ROWS // (NC * NS)
    NB = RW // CHD
    RB = NP // NS

    @functools.partial(
        pl.kernel,
        out_type=jax.ShapeDtypeStruct((NC * NP, D), jnp.float32),
        mesh=_sc_mesh(),
        scratch_types=[
            pltpu.VMEM((2, CHD, LANES), jnp.int32),
            pltpu.VMEM((LANES, D), jnp.float32),
            pltpu.VMEM_SHARED((NP, D), jnp.float32),
            pltpu.SemaphoreType.DMA,
            pltpu.SemaphoreType.DMA,
        ],
        compiler_params=_SC_PARAMS,
    )
    def deg_kernel(dst2_hbm, ones_hbm, z_hbm, out_hbm, idx, ones_v, acc,
                   lsem, ssem):
        c = lax.axis_index("c")
        s = lax.axis_index("s")
        wid = s * NC + c
        pltpu.sync_copy(z_hbm.at[pl.ds(s * RB, RB)], acc.at[pl.ds(s * RB, RB)])
        pltpu.sync_copy(ones_hbm, ones_v)
        plsc.subcore_barrier()

        base = wid * RW
        pltpu.sync_copy(dst2_hbm.at[pl.ds(base, CHD)], idx.at[0])

        @pl.loop(0, NB // 2)
        def _(t):
            for p in (0, 1):
                b = 2 * t + p
                q = 1 - p

                @pl.when(b > 0)
                def _():
                    for j in range(CHD):
                        pltpu.make_async_copy(
                            ones_v, acc.at[idx.at[q, j]], ssem
                        ).wait()

                @pl.when(b + 1 < NB)
                def _():
                    pltpu.async_copy(
                        dst2_hbm.at[pl.ds(base + (b + 1) * CHD, CHD)],
                        idx.at[q], lsem,
                    )

                for j in range(CHD):
                    pltpu.async_copy(
                        ones_v, acc.at[idx.at[p, j]], ssem, add=True
                    )

                @pl.when(b + 1 < NB)
                def _():
                    pltpu.make_async_copy(
                        dst2_hbm.at[pl.ds(0, CHD)], idx.at[q], lsem
                    ).wait()

        for j in range(CHD):
            pltpu.make_async_copy(ones_v, acc.at[idx.at[1, j]], ssem).wait()

        plsc.subcore_barrier()
        pltpu.sync_copy(
            acc.at[pl.ds(s * RB, RB)], out_hbm.at[pl.ds(c * NP + s * RB, RB)]
        )

    return deg_kernel


def _make_agg_kernel(NP, ROWS, D):
    RW = ROWS // (NC * NS)
    NB = RW // CHA
    RB = NP // NS

    @functools.partial(
        pl.kernel,
        out_type=jax.ShapeDtypeStruct((NC * NP, D), jnp.float32),
        mesh=_sc_mesh(),
        scratch_types=[
            pltpu.VMEM((2, CHA, LANES), jnp.int32),
            pltpu.VMEM((2, CHA, LANES), jnp.int32),
            pltpu.VMEM((2, CHA, LANES, D), jnp.float32),
            pltpu.VMEM_SHARED((NP, D), jnp.float32),
            pltpu.SemaphoreType.DMA,
            pltpu.SemaphoreType.DMA,
            pltpu.SemaphoreType.DMA,
        ],
        compiler_params=_SC_PARAMS,
    )
    def agg_kernel(src2_hbm, dst2_hbm, u_hbm, z_hbm, out_hbm,
                   isrc, idst, rows, acc, lsem, gsem, ssem):
        c = lax.axis_index("c")
        s = lax.axis_index("s")
        wid = s * NC + c
        pltpu.sync_copy(z_hbm.at[pl.ds(s * RB, RB)], acc.at[pl.ds(s * RB, RB)])
        plsc.subcore_barrier()

        base = wid * RW
        pltpu.sync_copy(src2_hbm.at[pl.ds(base, CHA)], isrc.at[0])
        pltpu.sync_copy(dst2_hbm.at[pl.ds(base, CHA)], idst.at[0])
        for j in range(CHA):
            pltpu.async_copy(u_hbm.at[isrc.at[0, j]], rows.at[0, j], gsem)

        @pl.loop(0, NB // 2)
        def _(t):
            for p in (0, 1):
                b = 2 * t + p
                q = 1 - p

                @pl.when(b > 0)
                def _():
                    for j in range(CHA):
                        pltpu.make_async_copy(
                            rows.at[q, j], acc.at[idst.at[q, j]], ssem
                        ).wait()

                @pl.when(b + 1 < NB)
                def _():
                    pltpu.async_copy(
                        src2_hbm.at[pl.ds(base + (b + 1) * CHA, CHA)],
                        isrc.at[q], lsem,
                    )
                    pltpu.async_copy(
                        dst2_hbm.at[pl.ds(base + (b + 1) * CHA, CHA)],
                        idst.at[q], lsem,
                    )

                for j in range(CHA):
                    pltpu.make_async_copy(
                        u_hbm.at[isrc.at[p, j]], rows.at[p, j], gsem
                    ).wait()
                for j in range(CHA):
                    pltpu.async_copy(
                        rows.at[p, j], acc.at[idst.at[p, j]], ssem, add=True
                    )

                @pl.when(b + 1 < NB)
                def _():
                    pltpu.make_async_copy(
                        src2_hbm.at[pl.ds(0, CHA)], isrc.at[q], lsem
                    ).wait()
                    pltpu.make_async_copy(
                        dst2_hbm.at[pl.ds(0, CHA)], idst.at[q], lsem
                    ).wait()
                    for j in range(CHA):
                        pltpu.async_copy(
                            u_hbm.at[isrc.at[q, j]], rows.at[q, j], gsem
                        )

        for j in range(CHA):
            pltpu.make_async_copy(
                rows.at[1, j], acc.at[idst.at[1, j]], ssem
            ).wait()

        plsc.subcore_barrier()
        pltpu.sync_copy(
            acc.at[pl.ds(s * RB, RB)], out_hbm.at[pl.ds(c * NP + s * RB, RB)]
        )

    return agg_kernel


def _tc1_body(d0, d1, x, w, o):
    dv = lax.rsqrt(d0[...] + d1[...] + 1.0)
    o[...] = jnp.dot(x[...], w[...], preferred_element_type=jnp.float32) * dv


def _tc2_body(d0, d1, a0, a1, u1, w, b, o):
    dv = lax.rsqrt(d0[...] + d1[...] + 1.0)
    t = dv * (a0[...] + a1[...] + u1[...]) + b[...]
    r = jnp.maximum(t, 0.0)
    o[...] = jnp.dot(r, w[...], preferred_element_type=jnp.float32) * dv


def _tc3_body(d0, d1, a0, a1, u2, b, o):
    dv = lax.rsqrt(d0[...] + d1[...] + 1.0)
    o[...] = dv * (a0[...] + a1[...] + u2[...]) + b[...]


def _half(bq, off):
    return pl.BlockSpec((bq, LANES), lambda i, o=off: (i + o, 0))


def _mat(bq):
    return pl.BlockSpec((bq, LANES), lambda i: (i, 0))


def _full(shape):
    return pl.BlockSpec(shape, lambda i: tuple(0 for _ in shape))


def kernel(x, edge_index, W1, b1, W2, b2):
    N, IN_D = x.shape
    HID = W1.shape[1]
    OUT = W2.shape[1]
    E = edge_index.shape[1]
    REP = LANES // HID

    NP = _round_up(N + 1, NS * REP * 56)
    Q = NP * HID // LANES
    GRIDQ = 7
    BQ = Q // GRIDQ
    EP = _round_up(E, NC * NS * 2 * max(CHA, CHD) * LANES)
    ROWS = EP // LANES

    f32 = jnp.float32
    pad = jnp.full((2, EP - E), N, jnp.int32)
    ep = jnp.concatenate([edge_index.astype(jnp.int32), pad], axis=1)
    src2 = ep[0].reshape(ROWS, LANES)
    dst2 = ep[1].reshape(ROWS, LANES)
    zt = jnp.zeros((NP, HID), f32)
    ones_t = jnp.ones((LANES, HID), f32)
    x_pk = x.astype(f32).reshape(N * IN_D // LANES, LANES)
    eye8 = jnp.eye(REP, dtype=f32)
    W1bd = jnp.kron(eye8, W1.astype(f32))
    W2p = jnp.concatenate(
        [W2.astype(f32), jnp.zeros((HID, HID - OUT), f32)], axis=1)
    W2bd = jnp.kron(eye8, W2p)
    b1t = jnp.tile(b1.astype(f32), (REP,)).reshape(1, LANES)
    b2t = jnp.tile(
        jnp.concatenate([b2.astype(f32), jnp.zeros((HID - OUT,), f32)]),
        (REP,)).reshape(1, LANES)

    degb = _make_deg_kernel(NP, ROWS, HID)(dst2, ones_t, zt)
    degq = degb.reshape(NC * Q, LANES)

    u1 = pl.pallas_call(
        _tc1_body,
        grid=(GRIDQ,),
        in_specs=[_half(BQ, 0), _half(BQ, GRIDQ), _mat(BQ),
                  _full((LANES, LANES))],
        out_specs=_mat(BQ),
        out_shape=jax.ShapeDtypeStruct((Q, LANES), f32),
    )(degq, degq, x_pk, W1bd)

    agg1 = _make_agg_kernel(NP, ROWS, HID)(src2, dst2, u1.reshape(NP, HID), zt)
    agg1q = agg1.reshape(NC * Q, LANES)

    u2 = pl.pallas_call(
        _tc2_body,
        grid=(GRIDQ,),
        in_specs=[_half(BQ, 0), _half(BQ, GRIDQ),
                  _half(BQ, 0), _half(BQ, GRIDQ), _mat(BQ),
                  _full((LANES, LANES)), _full((1, LANES))],
        out_specs=_mat(BQ),
        out_shape=jax.ShapeDtypeStruct((Q, LANES), f32),
    )(degq, degq, agg1q, agg1q, u1, W2bd, b1t)

    agg2 = _make_agg_kernel(NP, ROWS, HID)(src2, dst2, u2.reshape(NP, HID), zt)
    agg2q = agg2.reshape(NC * Q, LANES)

    opk = pl.pallas_call(
        _tc3_body,
        grid=(GRIDQ,),
        in_specs=[_half(BQ, 0), _half(BQ, GRIDQ),
                  _half(BQ, 0), _half(BQ, GRIDQ), _mat(BQ),
                  _full((1, LANES))],
        out_specs=_mat(BQ),
        out_shape=jax.ShapeDtypeStruct((Q, LANES), f32),
    )(degq, degq, agg2q, agg2q, u2, b2t)

    return opk.reshape(NP, HID)[:N, :OUT]


# speedup vs baseline: 96.7471x; 1.1096x over previous
import functools

import jax
import jax.numpy as jnp
from jax import lax
from jax.experimental import pallas as pl
from jax.experimental.pallas import tpu as pltpu
from jax.experimental.pallas import tpu_sc as plsc

NC = 2
NS = 16
LANES = 128
CHA = 8
CHD = 4


def _round_up(a, b):
    return (a + b - 1) // b * b


def _sc_mesh():
    return plsc.VectorSubcoreMesh(
        core_axis_name="c", subcore_axis_name="s", num_cores=NC, num_subcores=NS
    )


_SC_PARAMS = pltpu.CompilerParams(use_tc_tiling_on_sc=False)


def _make_deg_kernel(NP, ROWS, D):
    RW = ROWS // (NC * NS)
    NB = RW // CHD
    RB = NP // NS

    @functools.partial(
        pl.kernel,
        out_type=jax.ShapeDtypeStruct((NC * NP, D), jnp.float32),
        mesh=_sc_mesh(),
        scratch_types=[
            pltpu.VMEM((2, CHD, LANES), jnp.int32),
            pltpu.VMEM((LANES, D), jnp.float32),
            pltpu.VMEM_SHARED((NP, D), jnp.float32),
            pltpu.SemaphoreType.DMA,
            pltpu.SemaphoreType.DMA,
        ],
        compiler_params=_SC_PARAMS,
    )
    def deg_kernel(dst2_hbm, ones_hbm, z_hbm, out_hbm, idx, ones_v, acc,
                   lsem, ssem):
        c = lax.axis_index("c")
        s = lax.axis_index("s")
        wid = s * NC + c
        pltpu.sync_copy(z_hbm.at[pl.ds(s * RB, RB)], acc.at[pl.ds(s * RB, RB)])
        pltpu.sync_copy(ones_hbm, ones_v)
        plsc.subcore_barrier()

        base = wid * RW
        pltpu.sync_copy(dst2_hbm.at[pl.ds(base, CHD)], idx.at[0])

        @pl.loop(0, NB // 2)
        def _(t):
            for p in (0, 1):
                b = 2 * t + p
                q = 1 - p

                @pl.when(b > 0)
                def _():
                    for j in range(CHD):
                        pltpu.make_async_copy(
                            ones_v, acc.at[idx.at[q, j]], ssem
                        ).wait()

                @pl.when(b + 1 < NB)
                def _():
                    pltpu.async_copy(
                        dst2_hbm.at[pl.ds(base + (b + 1) * CHD, CHD)],
                        idx.at[q], lsem,
                    )

                for j in range(CHD):
                    pltpu.async_copy(
                        ones_v, acc.at[idx.at[p, j]], ssem, add=True
                    )

                @pl.when(b + 1 < NB)
                def _():
                    pltpu.make_async_copy(
                        dst2_hbm.at[pl.ds(0, CHD)], idx.at[q], lsem
                    ).wait()

        for j in range(CHD):
            pltpu.make_async_copy(ones_v, acc.at[idx.at[1, j]], ssem).wait()

        plsc.subcore_barrier()
        pltpu.sync_copy(
            acc.at[pl.ds(s * RB, RB)], out_hbm.at[pl.ds(c * NP + s * RB, RB)]
        )

    return deg_kernel


def _make_agg_kernel(NP, ROWS, D, CH):
    RW = ROWS // (NC * NS)
    NB = RW // CH
    RB = NP // NS
    EB = CH * LANES

    @functools.partial(
        pl.kernel,
        out_type=jax.ShapeDtypeStruct((NC * NP, D), jnp.float32),
        mesh=_sc_mesh(),
        scratch_types=[
            pltpu.VMEM((EB,), jnp.int32),
            pltpu.VMEM((EB,), jnp.int32),
            pltpu.VMEM((2, CH, LANES), jnp.int32),
            pltpu.VMEM((EB, D), jnp.float32),
            pltpu.VMEM((EB, D), jnp.float32),
            pltpu.VMEM_SHARED((NP, D), jnp.float32),
            pltpu.SemaphoreType.DMA,
            pltpu.SemaphoreType.DMA,
            pltpu.SemaphoreType.DMA,
        ],
        compiler_params=_SC_PARAMS,
    )
    def agg_kernel(src1_hbm, dst2_hbm, u_hbm, z_hbm, out_hbm,
                   isrc0, isrc1, idst, rows0, rows1, acc, lsem, gsem, ssem):
        c = lax.axis_index("c")
        s = lax.axis_index("s")
        wid = s * NC + c
        pltpu.sync_copy(z_hbm.at[pl.ds(s * RB, RB)], acc.at[pl.ds(s * RB, RB)])
        plsc.subcore_barrier()

        base = wid * RW
        pltpu.sync_copy(src1_hbm.at[pl.ds(base * LANES, EB)], isrc0)
        pltpu.sync_copy(dst2_hbm.at[pl.ds(base, CH)], idst.at[0])
        pltpu.async_copy(u_hbm.at[isrc0], rows0, gsem)

        @pl.loop(0, NB // 2)
        def _(t):
            for p in (0, 1):
                b = 2 * t + p
                isp, isq = (isrc0, isrc1) if p == 0 else (isrc1, isrc0)
                rsp, rsq = (rows0, rows1) if p == 0 else (rows1, rows0)
                q = 1 - p

                @pl.when(b > 0)
                def _():
                    for j in range(CH):
                        pltpu.make_async_copy(
                            rsq.at[pl.ds(j * LANES, LANES)],
                            acc.at[idst.at[q, j]], ssem,
                        ).wait()

                @pl.when(b + 1 < NB)
                def _():
                    pltpu.async_copy(
                        src1_hbm.at[pl.ds((base + (b + 1) * CH) * LANES, EB)],
                        isq, lsem,
                    )
                    pltpu.async_copy(
                        dst2_hbm.at[pl.ds(base + (b + 1) * CH, CH)],
                        idst.at[q], lsem,
                    )

                pltpu.make_async_copy(u_hbm.at[isp], rsp, gsem).wait()
                for j in range(CH):
                    pltpu.async_copy(
                        rsp.at[pl.ds(j * LANES, LANES)],
                        acc.at[idst.at[p, j]], ssem, add=True,
                    )

                @pl.when(b + 1 < NB)
                def _():
                    pltpu.make_async_copy(
                        src1_hbm.at[pl.ds(0, EB)], isq, lsem
                    ).wait()
                    pltpu.make_async_copy(
                        dst2_hbm.at[pl.ds(0, CH)], idst.at[q], lsem
                    ).wait()
                    pltpu.async_copy(u_hbm.at[isq], rsq, gsem)

        for j in range(CH):
            pltpu.make_async_copy(
                rows1.at[pl.ds(j * LANES, LANES)],
                acc.at[idst.at[1, j]], ssem,
            ).wait()

        plsc.subcore_barrier()
        pltpu.sync_copy(
            acc.at[pl.ds(s * RB, RB)], out_hbm.at[pl.ds(c * NP + s * RB, RB)]
        )

    return agg_kernel


def _tc1_body(d0, d1, x, e16, w, o):
    s = d0[...] + d1[...]
    dv16 = lax.rsqrt(
        jnp.dot(s, e16[...], preferred_element_type=jnp.float32) + 1.0)
    o[...] = jnp.dot(x[...], w[...],
                     preferred_element_type=jnp.float32) * dv16


def _tc2_body(d0, d1, a0, a1, u1, e16, w, b, o):
    s = d0[...] + d1[...]
    dv16 = lax.rsqrt(
        jnp.dot(s, e16[...], preferred_element_type=jnp.float32) + 1.0)
    dv8 = lax.rsqrt(s + 1.0)
    t = dv16 * (a0[...] + a1[...] + u1[...]) + b[...]
    r = jnp.maximum(t, 0.0)
    o[...] = jnp.dot(r, w[...], preferred_element_type=jnp.float32) * dv8


def _tc3_body(d0, d1, c0, c1, u2, b, o):
    dv8 = lax.rsqrt(d0[...] + d1[...] + 1.0)
    o[...] = dv8 * (c0[...] + c1[...] + u2[...]) + b[...]


def _half(bh, w, off):
    return pl.BlockSpec((bh, w), lambda i, o=off: (i + o, 0))


def _mat(bh, w):
    return pl.BlockSpec((bh, w), lambda i: (i, 0))


def _full(shape):
    return pl.BlockSpec(shape, lambda i: tuple(0 for _ in shape))


def kernel(x, edge_index, W1, b1, W2, b2):
    N, IN_D = x.shape
    HID = W1.shape[1]
    OUT = W2.shape[1]
    E = edge_index.shape[1]

    NP = _round_up(N + 1, NS * 8 * 56)
    QR = NP // 16
    GRIDQ = 7
    BH = QR // GRIDQ
    EP = _round_up(E, NC * NS * 2 * max(CHA, CHD) * LANES)
    ROWS = EP // LANES

    f32 = jnp.float32
    pad = jnp.full((2, EP - E), N, jnp.int32)
    ep = jnp.concatenate([edge_index.astype(jnp.int32), pad], axis=1)
    src1 = ep[0]
    dst2 = ep[1].reshape(ROWS, LANES)
    z16 = jnp.zeros((NP, HID), f32)
    z8 = jnp.zeros((NP, OUT), f32)
    ones8 = jnp.ones((LANES, OUT), f32)
    x256 = x.astype(f32).reshape(N * IN_D // 256, 256)
    eye16 = jnp.eye(16, dtype=f32)
    E16 = jnp.kron(eye16, jnp.ones((OUT, HID), f32) / OUT)
    W1bd = jnp.kron(eye16, W1.astype(f32))
    W2bd = jnp.kron(eye16, W2.astype(f32))
    b1t = jnp.tile(b1.astype(f32), (16,)).reshape(1, 16 * HID)
    b2t = jnp.tile(b2.astype(f32), (16,)).reshape(1, 16 * OUT)

    degb = _make_deg_kernel(NP, ROWS, OUT)(dst2, ones8, z8)
    degq = degb.reshape(NC * QR, 16 * OUT)

    u1 = pl.pallas_call(
        _tc1_body,
        grid=(GRIDQ,),
        in_specs=[_half(BH, 128, 0), _half(BH, 128, GRIDQ), _mat(BH, 256),
                  _full((128, 256)), _full((256, 256))],
        out_specs=_mat(BH, 256),
        out_shape=jax.ShapeDtypeStruct((QR, 256), f32),
    )(degq, degq, x256, E16, W1bd)

    agg1 = _make_agg_kernel(NP, ROWS, HID, 4)(src1, dst2, u1.reshape(NP, HID),
                                           z16)
    agg1q = agg1.reshape(NC * QR, 16 * HID)

    u2 = pl.pallas_call(
        _tc2_body,
        grid=(GRIDQ,),
        in_specs=[_half(BH, 128, 0), _half(BH, 128, GRIDQ),
                  _half(BH, 256, 0), _half(BH, 256, GRIDQ), _mat(BH, 256),
                  _full((128, 256)), _full((256, 128)), _full((1, 256))],
        out_specs=_mat(BH, 128),
        out_shape=jax.ShapeDtypeStruct((QR, 128), f32),
    )(degq, degq, agg1q, agg1q, u1, E16, W2bd, b1t)

    agg2 = _make_agg_kernel(NP, ROWS, OUT, CHA)(src1, dst2, u2.reshape(NP, OUT), z8)
    agg2q = agg2.reshape(NC * QR, 16 * OUT)

    opk = pl.pallas_call(
        _tc3_body,
        grid=(GRIDQ,),
        in_specs=[_half(BH, 128, 0), _half(BH, 128, GRIDQ),
                  _half(BH, 128, 0), _half(BH, 128, GRIDQ), _mat(BH, 128),
                  _full((1, 128))],
        out_specs=_mat(BH, 128),
        out_shape=jax.ShapeDtypeStruct((QR, 128), f32),
    )(degq, degq, agg2q, agg2q, u2, b2t)

    return opk.reshape(NP, OUT)[:N]


# speedup vs baseline: 101.9300x; 1.0536x over previous
import functools

import jax
import jax.numpy as jnp
from jax import lax
from jax.experimental import pallas as pl
from jax.experimental.pallas import tpu as pltpu
from jax.experimental.pallas import tpu_sc as plsc

NC = 2
NS = 16
LANES = 128
CHA = 8
CHD = 8


def _round_up(a, b):
    return (a + b - 1) // b * b


def _sc_mesh():
    return plsc.VectorSubcoreMesh(
        core_axis_name="c", subcore_axis_name="s", num_cores=NC, num_subcores=NS
    )


_SC_PARAMS = pltpu.CompilerParams(use_tc_tiling_on_sc=False)


def _make_deg_kernel(NP, EP, D):
    EW = EP // (NC * NS)
    EB = CHD * LANES
    NB = EW // EB
    RB = NP // NS

    @functools.partial(
        pl.kernel,
        out_type=jax.ShapeDtypeStruct((NC * NP, D), jnp.float32),
        mesh=_sc_mesh(),
        scratch_types=[
            pltpu.VMEM((EB,), jnp.int32),
            pltpu.VMEM((EB,), jnp.int32),
            pltpu.VMEM((EB, D), jnp.float32),
            pltpu.VMEM_SHARED((NP, D), jnp.float32),
            pltpu.SemaphoreType.DMA,
            pltpu.SemaphoreType.DMA,
        ],
        compiler_params=_SC_PARAMS,
    )
    def deg_kernel(dst1_hbm, ones_hbm, z_hbm, out_hbm, idst0, idst1, ones_v,
                   acc, lsem, ssem):
        c = lax.axis_index("c")
        s = lax.axis_index("s")
        wid = s * NC + c
        pltpu.sync_copy(z_hbm.at[pl.ds(s * RB, RB)], acc.at[pl.ds(s * RB, RB)])
        pltpu.sync_copy(ones_hbm, ones_v)
        plsc.subcore_barrier()

        base = wid * EW
        pltpu.sync_copy(dst1_hbm.at[pl.ds(base, EB)], idst0)

        @pl.loop(0, NB // 2)
        def _(t):
            for p in (0, 1):
                b = 2 * t + p
                idsp, idsq = (idst0, idst1) if p == 0 else (idst1, idst0)

                @pl.when(b > 0)
                def _():
                    pltpu.make_async_copy(ones_v, acc.at[idsq], ssem).wait()

                @pl.when(b + 1 < NB)
                def _():
                    pltpu.async_copy(
                        dst1_hbm.at[pl.ds(base + (b + 1) * EB, EB)],
                        idsq, lsem,
                    )

                pltpu.async_copy(ones_v, acc.at[idsp], ssem, add=True)

                @pl.when(b + 1 < NB)
                def _():
                    pltpu.make_async_copy(
                        dst1_hbm.at[pl.ds(0, EB)], idsq, lsem
                    ).wait()

        pltpu.make_async_copy(ones_v, acc.at[idst1], ssem).wait()

        plsc.subcore_barrier()
        pltpu.sync_copy(
            acc.at[pl.ds(s * RB, RB)], out_hbm.at[pl.ds(c * NP + s * RB, RB)]
        )

    return deg_kernel


def _make_agg_kernel(NP, EP, D, CH):
    EW = EP // (NC * NS)
    EB = CH * LANES
    NB = EW // EB
    RB = NP // NS

    @functools.partial(
        pl.kernel,
        out_type=jax.ShapeDtypeStruct((NC * NP, D), jnp.float32),
        mesh=_sc_mesh(),
        scratch_types=[
            pltpu.VMEM((EB,), jnp.int32),
            pltpu.VMEM((EB,), jnp.int32),
            pltpu.VMEM((EB,), jnp.int32),
            pltpu.VMEM((EB,), jnp.int32),
            pltpu.VMEM((EB, D), jnp.float32),
            pltpu.VMEM((EB, D), jnp.float32),
            pltpu.VMEM_SHARED((NP, D), jnp.float32),
            pltpu.SemaphoreType.DMA,
            pltpu.SemaphoreType.DMA,
            pltpu.SemaphoreType.DMA,
        ],
        compiler_params=_SC_PARAMS,
    )
    def agg_kernel(src1_hbm, dst1_hbm, u_hbm, z_hbm, out_hbm,
                   isrc0, isrc1, idst0, idst1, rows0, rows1, acc,
                   lsem, gsem, ssem):
        c = lax.axis_index("c")
        s = lax.axis_index("s")
        wid = s * NC + c
        pltpu.sync_copy(z_hbm.at[pl.ds(s * RB, RB)], acc.at[pl.ds(s * RB, RB)])
        plsc.subcore_barrier()

        base = wid * EW
        pltpu.sync_copy(src1_hbm.at[pl.ds(base, EB)], isrc0)
        pltpu.sync_copy(dst1_hbm.at[pl.ds(base, EB)], idst0)
        pltpu.async_copy(u_hbm.at[isrc0], rows0, gsem)

        @pl.loop(0, NB // 2)
        def _(t):
            for p in (0, 1):
                b = 2 * t + p
                isp, isq = (isrc0, isrc1) if p == 0 else (isrc1, isrc0)
                idsp, idsq = (idst0, idst1) if p == 0 else (idst1, idst0)
                rsp, rsq = (rows0, rows1) if p == 0 else (rows1, rows0)

                @pl.when(b > 0)
                def _():
                    pltpu.make_async_copy(rsq, acc.at[idsq], ssem).wait()

                @pl.when(b + 1 < NB)
                def _():
                    pltpu.async_copy(
                        src1_hbm.at[pl.ds(base + (b + 1) * EB, EB)],
                        isq, lsem,
                    )
                    pltpu.async_copy(
                        dst1_hbm.at[pl.ds(base + (b + 1) * EB, EB)],
                        idsq, lsem,
                    )

                pltpu.make_async_copy(u_hbm.at[isp], rsp, gsem).wait()
                pltpu.async_copy(rsp, acc.at[idsp], ssem, add=True)

                @pl.when(b + 1 < NB)
                def _():
                    pltpu.make_async_copy(
                        src1_hbm.at[pl.ds(0, EB)], isq, lsem
                    ).wait()
                    pltpu.make_async_copy(
                        dst1_hbm.at[pl.ds(0, EB)], idsq, lsem
                    ).wait()
                    pltpu.async_copy(u_hbm.at[isq], rsq, gsem)

        pltpu.make_async_copy(rows1, acc.at[idst1], ssem).wait()

        plsc.subcore_barrier()
        pltpu.sync_copy(
            acc.at[pl.ds(s * RB, RB)], out_hbm.at[pl.ds(c * NP + s * RB, RB)]
        )

    return agg_kernel


def _tc1_body(d0, d1, x, e16, w, o):
    s = d0[...] + d1[...]
    dv16 = lax.rsqrt(
        jnp.dot(s, e16[...], preferred_element_type=jnp.float32) + 1.0)
    o[...] = jnp.dot(x[...], w[...],
                     preferred_element_type=jnp.float32) * dv16


def _tc2_body(d0, d1, a0, a1, u1, e16, w, b, o):
    s = d0[...] + d1[...]
    dv16 = lax.rsqrt(
        jnp.dot(s, e16[...], preferred_element_type=jnp.float32) + 1.0)
    dv8 = lax.rsqrt(s + 1.0)
    t = dv16 * (a0[...] + a1[...] + u1[...]) + b[...]
    r = jnp.maximum(t, 0.0)
    o[...] = jnp.dot(r, w[...], preferred_element_type=jnp.float32) * dv8


def _tc3_body(d0, d1, c0, c1, u2, b, o):
    dv8 = lax.rsqrt(d0[...] + d1[...] + 1.0)
    o[...] = dv8 * (c0[...] + c1[...] + u2[...]) + b[...]


def _half(bh, w, off):
    return pl.BlockSpec((bh, w), lambda i, o=off: (i + o, 0))


def _mat(bh, w):
    return pl.BlockSpec((bh, w), lambda i: (i, 0))


def _full(shape):
    return pl.BlockSpec(shape, lambda i: tuple(0 for _ in shape))


def kernel(x, edge_index, W1, b1, W2, b2):
    N, IN_D = x.shape
    HID = W1.shape[1]
    OUT = W2.shape[1]
    E = edge_index.shape[1]

    NP = _round_up(N + 1, NS * 8 * 56)
    QR = NP // 16
    GRIDQ = 7
    BH = QR // GRIDQ
    EP = _round_up(E, NC * NS * 2 * max(CHA, CHD) * LANES)

    f32 = jnp.float32
    pad = jnp.full((2, EP - E), N, jnp.int32)
    ep = jnp.concatenate([edge_index.astype(jnp.int32), pad], axis=1)
    src1 = ep[0]
    dst1 = ep[1]
    z16 = jnp.zeros((NP, HID), f32)
    z8 = jnp.zeros((NP, OUT), f32)
    ones8 = jnp.ones((CHD * LANES, OUT), f32)
    x256 = x.astype(f32).reshape(N * IN_D // 256, 256)
    eye16 = jnp.eye(16, dtype=f32)
    E16 = jnp.kron(eye16, jnp.ones((OUT, HID), f32) / OUT)
    W1bd = jnp.kron(eye16, W1.astype(f32))
    W2bd = jnp.kron(eye16, W2.astype(f32))
    b1t = jnp.tile(b1.astype(f32), (16,)).reshape(1, 16 * HID)
    b2t = jnp.tile(b2.astype(f32), (16,)).reshape(1, 16 * OUT)

    degb = _make_deg_kernel(NP, EP, OUT)(dst1, ones8, z8)
    degq = degb.reshape(NC * QR, 16 * OUT)

    u1 = pl.pallas_call(
        _tc1_body,
        grid=(GRIDQ,),
        in_specs=[_half(BH, 128, 0), _half(BH, 128, GRIDQ), _mat(BH, 256),
                  _full((128, 256)), _full((256, 256))],
        out_specs=_mat(BH, 256),
        out_shape=jax.ShapeDtypeStruct((QR, 256), f32),
    )(degq, degq, x256, E16, W1bd)

    agg1 = _make_agg_kernel(NP, EP, HID, 4)(src1, dst1, u1.reshape(NP, HID),
                                            z16)
    agg1q = agg1.reshape(NC * QR, 16 * HID)

    u2 = pl.pallas_call(
        _tc2_body,
        grid=(GRIDQ,),
        in_specs=[_half(BH, 128, 0), _half(BH, 128, GRIDQ),
                  _half(BH, 256, 0), _half(BH, 256, GRIDQ), _mat(BH, 256),
                  _full((128, 256)), _full((256, 128)), _full((1, 256))],
        out_specs=_mat(BH, 128),
        out_shape=jax.ShapeDtypeStruct((QR, 128), f32),
    )(degq, degq, agg1q, agg1q, u1, E16, W2bd, b1t)

    agg2 = _make_agg_kernel(NP, EP, OUT, CHA)(src1, dst1, u2.reshape(NP, OUT),
                                              z8)
    agg2q = agg2.reshape(NC * QR, 16 * OUT)

    opk = pl.pallas_call(
        _tc3_body,
        grid=(GRIDQ,),
        in_specs=[_half(BH, 128, 0), _half(BH, 128, GRIDQ),
                  _half(BH, 128, 0), _half(BH, 128, GRIDQ), _mat(BH, 128),
                  _full((1, 128))],
        out_specs=_mat(BH, 128),
        out_shape=jax.ShapeDtypeStruct((QR, 128), f32),
    )(degq, degq, agg2q, agg2q, u2, b2t)

    return opk.reshape(NP, OUT)[:N]


# speedup vs baseline: 111.1562x; 1.0905x over previous
import functools

import jax
import jax.numpy as jnp
from jax import lax
from jax.experimental import pallas as pl
from jax.experimental.pallas import tpu as pltpu
from jax.experimental.pallas import tpu_sc as plsc

NC = 2
NS = 16
EB1 = 400
EB2 = 1000


def _round_up(a, b):
    return (a + b - 1) // b * b


def _sc_mesh():
    return plsc.VectorSubcoreMesh(
        core_axis_name="c", subcore_axis_name="s", num_cores=NC, num_subcores=NS
    )


_SC_PARAMS = pltpu.CompilerParams(use_tc_tiling_on_sc=False)


def _make_deg_kernel(NP, EP, D, EB):
    EW = EP // (NC * NS)
    NB = EW // EB
    RB = NP // NS

    @functools.partial(
        pl.kernel,
        out_type=jax.ShapeDtypeStruct((NC * NP, D), jnp.float32),
        mesh=_sc_mesh(),
        scratch_types=[
            pltpu.VMEM((EB,), jnp.int32),
            pltpu.VMEM((EB,), jnp.int32),
            pltpu.VMEM((EB, D), jnp.float32),
            pltpu.VMEM_SHARED((NP, D), jnp.float32),
            pltpu.SemaphoreType.DMA,
            pltpu.SemaphoreType.DMA,
        ],
        compiler_params=_SC_PARAMS,
    )
    def deg_kernel(dst1_hbm, ones_hbm, z_hbm, out_hbm, idst0, idst1, ones_v,
                   acc, lsem, ssem):
        c = lax.axis_index("c")
        s = lax.axis_index("s")
        wid = s * NC + c
        pltpu.sync_copy(z_hbm.at[pl.ds(s * RB, RB)], acc.at[pl.ds(s * RB, RB)])
        pltpu.sync_copy(ones_hbm, ones_v)
        plsc.subcore_barrier()

        base = wid * EW
        pltpu.sync_copy(dst1_hbm.at[pl.ds(base, EB)], idst0)

        @pl.loop(0, NB // 2)
        def _(t):
            for p in (0, 1):
                b = 2 * t + p
                idsp, idsq = (idst0, idst1) if p == 0 else (idst1, idst0)

                @pl.when(b > 0)
                def _():
                    pltpu.make_async_copy(ones_v, acc.at[idsq], ssem).wait()

                @pl.when(b + 1 < NB)
                def _():
                    pltpu.async_copy(
                        dst1_hbm.at[pl.ds(base + (b + 1) * EB, EB)],
                        idsq, lsem,
                    )

                pltpu.async_copy(ones_v, acc.at[idsp], ssem, add=True)

                @pl.when(b + 1 < NB)
                def _():
                    pltpu.make_async_copy(
                        dst1_hbm.at[pl.ds(0, EB)], idsq, lsem
                    ).wait()

        pltpu.make_async_copy(ones_v, acc.at[idst1], ssem).wait()

        plsc.subcore_barrier()
        pltpu.sync_copy(
            acc.at[pl.ds(s * RB, RB)], out_hbm.at[pl.ds(c * NP + s * RB, RB)]
        )

    return deg_kernel


def _make_agg_kernel(NP, EP, D, EB):
    EW = EP // (NC * NS)
    NB = EW // EB
    RB = NP // NS

    @functools.partial(
        pl.kernel,
        out_type=jax.ShapeDtypeStruct((NC * NP, D), jnp.float32),
        mesh=_sc_mesh(),
        scratch_types=[
            pltpu.VMEM((EB,), jnp.int32),
            pltpu.VMEM((EB,), jnp.int32),
            pltpu.VMEM((EB,), jnp.int32),
            pltpu.VMEM((EB,), jnp.int32),
            pltpu.VMEM((EB, D), jnp.float32),
            pltpu.VMEM((EB, D), jnp.float32),
            pltpu.VMEM_SHARED((NP, D), jnp.float32),
            pltpu.SemaphoreType.DMA,
            pltpu.SemaphoreType.DMA,
            pltpu.SemaphoreType.DMA,
        ],
        compiler_params=_SC_PARAMS,
    )
    def agg_kernel(src1_hbm, dst1_hbm, u_hbm, z_hbm, out_hbm,
                   isrc0, isrc1, idst0, idst1, rows0, rows1, acc,
                   lsem, gsem, ssem):
        c = lax.axis_index("c")
        s = lax.axis_index("s")
        wid = s * NC + c
        pltpu.sync_copy(z_hbm.at[pl.ds(s * RB, RB)], acc.at[pl.ds(s * RB, RB)])
        plsc.subcore_barrier()

        base = wid * EW
        pltpu.sync_copy(src1_hbm.at[pl.ds(base, EB)], isrc0)
        pltpu.sync_copy(dst1_hbm.at[pl.ds(base, EB)], idst0)
        pltpu.async_copy(u_hbm.at[isrc0], rows0, gsem)

        @pl.loop(0, NB // 2)
        def _(t):
            for p in (0, 1):
                b = 2 * t + p
                isp, isq = (isrc0, isrc1) if p == 0 else (isrc1, isrc0)
                idsp, idsq = (idst0, idst1) if p == 0 else (idst1, idst0)
                rsp, rsq = (rows0, rows1) if p == 0 else (rows1, rows0)

                @pl.when(b > 0)
                def _():
                    pltpu.make_async_copy(rsq, acc.at[idsq], ssem).wait()

                @pl.when(b + 1 < NB)
                def _():
                    pltpu.async_copy(
                        src1_hbm.at[pl.ds(base + (b + 1) * EB, EB)],
                        isq, lsem,
                    )
                    pltpu.async_copy(
                        dst1_hbm.at[pl.ds(base + (b + 1) * EB, EB)],
                        idsq, lsem,
                    )

                pltpu.make_async_copy(u_hbm.at[isp], rsp, gsem).wait()
                pltpu.async_copy(rsp, acc.at[idsp], ssem, add=True)

                @pl.when(b + 1 < NB)
                def _():
                    pltpu.make_async_copy(
                        src1_hbm.at[pl.ds(0, EB)], isq, lsem
                    ).wait()
                    pltpu.make_async_copy(
                        dst1_hbm.at[pl.ds(0, EB)], idsq, lsem
                    ).wait()
                    pltpu.async_copy(u_hbm.at[isq], rsq, gsem)

        pltpu.make_async_copy(rows1, acc.at[idst1], ssem).wait()

        plsc.subcore_barrier()
        pltpu.sync_copy(
            acc.at[pl.ds(s * RB, RB)], out_hbm.at[pl.ds(c * NP + s * RB, RB)]
        )

    return agg_kernel


def _tc1_body(d0, d1, x, e16, w, o):
    s = d0[...] + d1[...]
    dv16 = lax.rsqrt(
        jnp.dot(s, e16[...], preferred_element_type=jnp.float32) + 1.0)
    o[...] = jnp.dot(x[...], w[...],
                     preferred_element_type=jnp.float32) * dv16


def _tc2_body(d0, d1, a0, a1, u1, e16, w, b, o):
    s = d0[...] + d1[...]
    dv16 = lax.rsqrt(
        jnp.dot(s, e16[...], preferred_element_type=jnp.float32) + 1.0)
    dv8 = lax.rsqrt(s + 1.0)
    t = dv16 * (a0[...] + a1[...] + u1[...]) + b[...]
    r = jnp.maximum(t, 0.0)
    o[...] = jnp.dot(r, w[...], preferred_element_type=jnp.float32) * dv8


def _tc3_body(d0, d1, c0, c1, u2, b, o):
    dv8 = lax.rsqrt(d0[...] + d1[...] + 1.0)
    o[...] = dv8 * (c0[...] + c1[...] + u2[...]) + b[...]


def _half(bh, w, off):
    return pl.BlockSpec((bh, w), lambda i, o=off: (i + o, 0))


def _mat(bh, w):
    return pl.BlockSpec((bh, w), lambda i: (i, 0))


def _full(shape):
    return pl.BlockSpec(shape, lambda i: tuple(0 for _ in shape))


def kernel(x, edge_index, W1, b1, W2, b2):
    N, IN_D = x.shape
    HID = W1.shape[1]
    OUT = W2.shape[1]
    E = edge_index.shape[1]

    NP = _round_up(N + 1, NS * 8 * 56)
    QR = NP // 16
    GRIDQ = 7
    BH = QR // GRIDQ
    EP = _round_up(E, NC * NS * 2 * EB1 * EB2 // 200)

    f32 = jnp.float32
    ei = edge_index.astype(jnp.int32)
    if EP > E:
        ei = jnp.concatenate([ei, jnp.full((2, EP - E), N, jnp.int32)], axis=1)
    src1 = ei[0]
    dst1 = ei[1]
    z16 = jnp.zeros((NP, HID), f32)
    z8 = jnp.zeros((NP, OUT), f32)
    ones8 = jnp.ones((EB2, OUT), f32)
    x256 = x.astype(f32).reshape(N * IN_D // 256, 256)
    eye16 = jnp.eye(16, dtype=f32)
    E16 = jnp.kron(eye16, jnp.ones((OUT, HID), f32) / OUT)
    W1bd = jnp.kron(eye16, W1.astype(f32))
    W2bd = jnp.kron(eye16, W2.astype(f32))
    b1t = jnp.tile(b1.astype(f32), (16,)).reshape(1, 16 * HID)
    b2t = jnp.tile(b2.astype(f32), (16,)).reshape(1, 16 * OUT)

    degb = _make_deg_kernel(NP, EP, OUT, EB2)(dst1, ones8, z8)
    degq = degb.reshape(NC * QR, 16 * OUT)

    u1 = pl.pallas_call(
        _tc1_body,
        grid=(GRIDQ,),
        in_specs=[_half(BH, 128, 0), _half(BH, 128, GRIDQ), _mat(BH, 256),
                  _full((128, 256)), _full((256, 256))],
        out_specs=_mat(BH, 256),
        out_shape=jax.ShapeDtypeStruct((QR, 256), f32),
    )(degq, degq, x256, E16, W1bd)

    agg1 = _make_agg_kernel(NP, EP, HID, EB1)(src1, dst1, u1.reshape(NP, HID),
                                            z16)
    agg1q = agg1.reshape(NC * QR, 16 * HID)

    u2 = pl.pallas_call(
        _tc2_body,
        grid=(GRIDQ,),
        in_specs=[_half(BH, 128, 0), _half(BH, 128, GRIDQ),
                  _half(BH, 256, 0), _half(BH, 256, GRIDQ), _mat(BH, 256),
                  _full((128, 256)), _full((256, 128)), _full((1, 256))],
        out_specs=_mat(BH, 128),
        out_shape=jax.ShapeDtypeStruct((QR, 128), f32),
    )(degq, degq, agg1q, agg1q, u1, E16, W2bd, b1t)

    agg2 = _make_agg_kernel(NP, EP, OUT, EB2)(src1, dst1, u2.reshape(NP, OUT),
                                              z8)
    agg2q = agg2.reshape(NC * QR, 16 * OUT)

    opk = pl.pallas_call(
        _tc3_body,
        grid=(GRIDQ,),
        in_specs=[_half(BH, 128, 0), _half(BH, 128, GRIDQ),
                  _half(BH, 128, 0), _half(BH, 128, GRIDQ), _mat(BH, 128),
                  _full((1, 128))],
        out_specs=_mat(BH, 128),
        out_shape=jax.ShapeDtypeStruct((QR, 128), f32),
    )(degq, degq, agg2q, agg2q, u2, b2t)

    return opk[:N * OUT // 128].reshape(N, OUT)


# speedup vs baseline: 119.9686x; 1.0793x over previous
import functools

import jax
import jax.numpy as jnp
from jax import lax
from jax.experimental import pallas as pl
from jax.experimental.pallas import tpu as pltpu
from jax.experimental.pallas import tpu_sc as plsc

NC = 2
NS = 16
EB1 = 400
EB2 = 2000


def _round_up(a, b):
    return (a + b - 1) // b * b


def _sc_mesh():
    return plsc.VectorSubcoreMesh(
        core_axis_name="c", subcore_axis_name="s", num_cores=NC, num_subcores=NS
    )


_SC_PARAMS = pltpu.CompilerParams(use_tc_tiling_on_sc=False)


def _make_deg_kernel(NP, EP, D, EB):
    EW = EP // (NC * NS)
    NB = EW // EB
    RB = NP // NS

    @functools.partial(
        pl.kernel,
        out_type=jax.ShapeDtypeStruct((NC * NP, D), jnp.float32),
        mesh=_sc_mesh(),
        scratch_types=[
            pltpu.VMEM((EB,), jnp.int32),
            pltpu.VMEM((EB,), jnp.int32),
            pltpu.VMEM((EB, D), jnp.float32),
            pltpu.VMEM_SHARED((NP, D), jnp.float32),
            pltpu.SemaphoreType.DMA,
            pltpu.SemaphoreType.DMA,
        ],
        compiler_params=_SC_PARAMS,
    )
    def deg_kernel(dst1_hbm, ones_hbm, z_hbm, out_hbm, idst0, idst1, ones_v,
                   acc, lsem, ssem):
        c = lax.axis_index("c")
        s = lax.axis_index("s")
        wid = s * NC + c
        pltpu.sync_copy(z_hbm.at[pl.ds(s * RB, RB)], acc.at[pl.ds(s * RB, RB)])
        pltpu.sync_copy(ones_hbm, ones_v)
        plsc.subcore_barrier()

        base = wid * EW
        pltpu.sync_copy(dst1_hbm.at[pl.ds(base, EB)], idst0)

        @pl.loop(0, NB // 2)
        def _(t):
            for p in (0, 1):
                b = 2 * t + p
                idsp, idsq = (idst0, idst1) if p == 0 else (idst1, idst0)

                @pl.when(b > 0)
                def _():
                    pltpu.make_async_copy(ones_v, acc.at[idsq], ssem).wait()

                @pl.when(b + 1 < NB)
                def _():
                    pltpu.async_copy(
                        dst1_hbm.at[pl.ds(base + (b + 1) * EB, EB)],
                        idsq, lsem,
                    )

                pltpu.async_copy(ones_v, acc.at[idsp], ssem, add=True)

                @pl.when(b + 1 < NB)
                def _():
                    pltpu.make_async_copy(
                        dst1_hbm.at[pl.ds(0, EB)], idsq, lsem
                    ).wait()

        pltpu.make_async_copy(ones_v, acc.at[idst1], ssem).wait()

        plsc.subcore_barrier()
        pltpu.sync_copy(
            acc.at[pl.ds(s * RB, RB)], out_hbm.at[pl.ds(c * NP + s * RB, RB)]
        )

    return deg_kernel


def _make_agg_kernel(NP, EP, D, EB):
    EW = EP // (NC * NS)
    NB = EW // EB
    RB = NP // NS

    @functools.partial(
        pl.kernel,
        out_type=jax.ShapeDtypeStruct((NC * NP, D), jnp.float32),
        mesh=_sc_mesh(),
        scratch_types=[
            pltpu.VMEM((EB,), jnp.int32),
            pltpu.VMEM((EB,), jnp.int32),
            pltpu.VMEM((EB,), jnp.int32),
            pltpu.VMEM((EB,), jnp.int32),
            pltpu.VMEM((EB, D), jnp.float32),
            pltpu.VMEM((EB, D), jnp.float32),
            pltpu.VMEM_SHARED((NP, D), jnp.float32),
            pltpu.SemaphoreType.DMA,
            pltpu.SemaphoreType.DMA,
            pltpu.SemaphoreType.DMA,
        ],
        compiler_params=_SC_PARAMS,
    )
    def agg_kernel(src1_hbm, dst1_hbm, u_hbm, z_hbm, out_hbm,
                   isrc0, isrc1, idst0, idst1, rows0, rows1, acc,
                   lsem, gsem, ssem):
        c = lax.axis_index("c")
        s = lax.axis_index("s")
        wid = s * NC + c
        pltpu.sync_copy(z_hbm.at[pl.ds(s * RB, RB)], acc.at[pl.ds(s * RB, RB)])
        plsc.subcore_barrier()

        base = wid * EW
        pltpu.sync_copy(src1_hbm.at[pl.ds(base, EB)], isrc0)
        pltpu.sync_copy(dst1_hbm.at[pl.ds(base, EB)], idst0)
        pltpu.async_copy(u_hbm.at[isrc0], rows0, gsem)

        @pl.loop(0, NB // 2)
        def _(t):
            for p in (0, 1):
                b = 2 * t + p
                isp, isq = (isrc0, isrc1) if p == 0 else (isrc1, isrc0)
                idsp, idsq = (idst0, idst1) if p == 0 else (idst1, idst0)
                rsp, rsq = (rows0, rows1) if p == 0 else (rows1, rows0)

                @pl.when(b > 0)
                def _():
                    pltpu.make_async_copy(rsq, acc.at[idsq], ssem).wait()

                @pl.when(b + 1 < NB)
                def _():
                    pltpu.async_copy(
                        src1_hbm.at[pl.ds(base + (b + 1) * EB, EB)],
                        isq, lsem,
                    )
                    pltpu.async_copy(
                        dst1_hbm.at[pl.ds(base + (b + 1) * EB, EB)],
                        idsq, lsem,
                    )

                pltpu.make_async_copy(u_hbm.at[isp], rsp, gsem).wait()
                pltpu.async_copy(rsp, acc.at[idsp], ssem, add=True)

                @pl.when(b + 1 < NB)
                def _():
                    pltpu.make_async_copy(
                        src1_hbm.at[pl.ds(0, EB)], isq, lsem
                    ).wait()
                    pltpu.make_async_copy(
                        dst1_hbm.at[pl.ds(0, EB)], idsq, lsem
                    ).wait()
                    pltpu.async_copy(u_hbm.at[isq], rsq, gsem)

        pltpu.make_async_copy(rows1, acc.at[idst1], ssem).wait()

        plsc.subcore_barrier()
        pltpu.sync_copy(
            acc.at[pl.ds(s * RB, RB)], out_hbm.at[pl.ds(c * NP + s * RB, RB)]
        )

    return agg_kernel


def _tc1_body(d0, d1, x, e16, w, o):
    s = d0[...] + d1[...]
    dv16 = lax.rsqrt(
        jnp.dot(s, e16[...], preferred_element_type=jnp.float32) + 1.0)
    o[...] = jnp.dot(x[...], w[...],
                     preferred_element_type=jnp.float32) * dv16


def _tc2_body(d0, d1, a0, a1, u1, e16, w, b, o):
    s = d0[...] + d1[...]
    dv16 = lax.rsqrt(
        jnp.dot(s, e16[...], preferred_element_type=jnp.float32) + 1.0)
    dv8 = lax.rsqrt(s + 1.0)
    t = dv16 * (a0[...] + a1[...] + u1[...]) + b[...]
    r = jnp.maximum(t, 0.0)
    o[...] = jnp.dot(r, w[...], preferred_element_type=jnp.float32) * dv8


def _tc3_body(d0, d1, c0, c1, u2, b, o):
    dv8 = lax.rsqrt(d0[...] + d1[...] + 1.0)
    o[...] = dv8 * (c0[...] + c1[...] + u2[...]) + b[...]


def _half(bh, w, off):
    return pl.BlockSpec((bh, w), lambda i, o=off: (i + o, 0))


def _mat(bh, w):
    return pl.BlockSpec((bh, w), lambda i: (i, 0))


def _full(shape):
    return pl.BlockSpec(shape, lambda i: tuple(0 for _ in shape))


def kernel(x, edge_index, W1, b1, W2, b2):
    N, IN_D = x.shape
    HID = W1.shape[1]
    OUT = W2.shape[1]
    E = edge_index.shape[1]

    NP = _round_up(N + 1, NS * 8 * 56)
    QR = NP // 16
    GRIDQ = 7
    BH = QR // GRIDQ
    EP = _round_up(E, NC * NS * 2 * EB1 * EB2 // 400)

    f32 = jnp.float32
    ei = edge_index.astype(jnp.int32)
    if EP > E:
        ei = jnp.concatenate([ei, jnp.full((2, EP - E), N, jnp.int32)], axis=1)
    src1 = ei[0]
    dst1 = ei[1]
    z16 = jnp.zeros((NP, HID), f32)
    z8 = jnp.zeros((NP, OUT), f32)
    ones8 = jnp.ones((EB2, OUT), f32)
    x256 = x.astype(f32).reshape(N * IN_D // 256, 256)
    eye16 = jnp.eye(16, dtype=f32)
    E16 = jnp.kron(eye16, jnp.ones((OUT, HID), f32) / OUT)
    W1bd = jnp.kron(eye16, W1.astype(f32))
    W2bd = jnp.kron(eye16, W2.astype(f32))
    b1t = jnp.tile(b1.astype(f32), (16,)).reshape(1, 16 * HID)
    b2t = jnp.tile(b2.astype(f32), (16,)).reshape(1, 16 * OUT)

    degb = _make_deg_kernel(NP, EP, OUT, EB2)(dst1, ones8, z8)
    degq = degb.reshape(NC * QR, 16 * OUT)

    u1 = pl.pallas_call(
        _tc1_body,
        grid=(GRIDQ,),
        in_specs=[_half(BH, 128, 0), _half(BH, 128, GRIDQ), _mat(BH, 256),
                  _full((128, 256)), _full((256, 256))],
        out_specs=_mat(BH, 256),
        out_shape=jax.ShapeDtypeStruct((QR, 256), f32),
    )(degq, degq, x256, E16, W1bd)

    agg1 = _make_agg_kernel(NP, EP, HID, EB1)(src1, dst1, u1.reshape(NP, HID),
                                            z16)
    agg1q = agg1.reshape(NC * QR, 16 * HID)

    u2 = pl.pallas_call(
        _tc2_body,
        grid=(GRIDQ,),
        in_specs=[_half(BH, 128, 0), _half(BH, 128, GRIDQ),
                  _half(BH, 256, 0), _half(BH, 256, GRIDQ), _mat(BH, 256),
                  _full((128, 256)), _full((256, 128)), _full((1, 256))],
        out_specs=_mat(BH, 128),
        out_shape=jax.ShapeDtypeStruct((QR, 128), f32),
    )(degq, degq, agg1q, agg1q, u1, E16, W2bd, b1t)

    agg2 = _make_agg_kernel(NP, EP, OUT, EB2)(src1, dst1, u2.reshape(NP, OUT),
                                              z8)
    agg2q = agg2.reshape(NC * QR, 16 * OUT)

    opk = pl.pallas_call(
        _tc3_body,
        grid=(GRIDQ,),
        in_specs=[_half(BH, 128, 0), _half(BH, 128, GRIDQ),
                  _half(BH, 128, 0), _half(BH, 128, GRIDQ), _mat(BH, 128),
                  _full((1, 128))],
        out_specs=_mat(BH, 128),
        out_shape=jax.ShapeDtypeStruct((QR, 128), f32),
    )(degq, degq, agg2q, agg2q, u2, b2t)

    return opk[:N * OUT // 128].reshape(N, OUT)
